# baseline (speedup 1.0000x reference)
import functools

import jax
import jax.numpy as jnp
from jax import lax
from jax.experimental import pallas as pl
from jax.experimental.pallas import tpu as pltpu
from jax.experimental.pallas import tpu_sc as plsc

EDGE_DIM = 64
PAIR_DIM = 2 * EDGE_DIM
NUM_WORKERS = 32
LANES = 16
MAX_GATHER_CHUNK = 112


def _round_up(x, m):
    return (x + m - 1) // m * m


@functools.lru_cache(maxsize=None)
def _build_sc_call(seq_len):
    b0 = 2 * (seq_len - 1)
    b1 = b0 + 2 * (seq_len - 2)
    b2 = b1 + 2 * (seq_len - 3)
    num_edges = b2 + seq_len
    assert num_edges % 2 == 0
    num_pairs = num_edges // 2

    ppw = _round_up(-(-num_pairs // NUM_WORKERS), LANES)
    p_pad = ppw * NUM_WORKERS
    epw = 2 * ppw
    e_pad = 2 * p_pad

    chunk = MAX_GATHER_CHUNK
    while ppw % chunk:
        chunk -= LANES
    n_chunks = ppw // chunk
    cpr = chunk // LANES

    mesh = plsc.VectorSubcoreMesh(core_axis_name="c", subcore_axis_name="s")

    @functools.partial(
        pl.kernel,
        mesh=mesh,
        out_type=[
            jax.ShapeDtypeStruct((2 * e_pad,), jnp.int32),
            jax.ShapeDtypeStruct((num_pairs, PAIR_DIM), jnp.float32),
        ],
        scratch_types=[
            pltpu.VMEM((n_chunks, chunk), jnp.int32),
            pltpu.VMEM((n_chunks, chunk), jnp.int32),
            pltpu.VMEM((ppw, PAIR_DIM), jnp.float32),
            pltpu.VMEM((epw,), jnp.int32),
            pltpu.VMEM((epw,), jnp.int32),
            pltpu.SemaphoreType.DMA,
        ],
    )
    def sc_kernel(
        emb_hbm, ei_hbm, attr_hbm, idx_v, didx_v, rows_v, src_v, dst_v, sem
    ):
        wid = lax.axis_index("s") * 2 + lax.axis_index("c")
        pair_base = wid * ppw
        edge_base = wid * epw

        for j in range(ppw // LANES):
            pc = pair_base + j * LANES + lax.iota(jnp.int32, LANES)
            c = 2 * pc
            t = jnp.where(
                c < b0, 0, jnp.where(c < b1, 1, jnp.where(c < b2, 2, 0))
            )
            idx_v[j // cpr, pl.ds((j % cpr) * LANES, LANES)] = t
            didx_v[j // cpr, pl.ds((j % cpr) * LANES, LANES)] = jnp.minimum(
                pc, num_pairs - 1
            )

        gathers = [
            pltpu.async_copy(
                emb_hbm.at[idx_v.at[ch]],
                rows_v.at[pl.ds(ch * chunk, chunk)],
                sem,
            )
            for ch in range(n_chunks)
        ]

        for j in range(epw // LANES):
            c = edge_base + j * LANES + lax.iota(jnp.int32, LANES)
            in0 = c < b0
            in1 = c < b1
            in2 = c < b2
            s = jnp.where(in0, 0, jnp.where(in1, b0, jnp.where(in2, b1, b2)))
            w = jnp.where(in0, 1, jnp.where(in1, 2, jnp.where(in2, 3, 0)))
            local = c - s
            k = local >> 1
            p = local & 1
            src = jnp.where(in2, k + p * w, local)
            dst = jnp.where(in2, k + (1 - p) * w, local)
            src_v[pl.ds(j * LANES, LANES)] = src
            dst_v[pl.ds(j * LANES, LANES)] = dst

        pltpu.sync_copy(src_v, ei_hbm.at[pl.ds(edge_base, epw)])
        pltpu.sync_copy(dst_v, ei_hbm.at[pl.ds(e_pad + edge_base, epw)])

        for cp in gathers:
            cp.wait()

        @pl.when(wid < NUM_WORKERS - 1)
        def _():
            pltpu.sync_copy(rows_v, attr_hbm.at[pl.ds(pair_base, ppw)])

        @pl.when(wid == NUM_WORKERS - 1)
        def _():
            scatters = [
                pltpu.async_copy(
                    rows_v.at[pl.ds(ch * chunk, chunk)],
                    attr_hbm.at[didx_v.at[ch]],
                    sem,
                )
                for ch in range(n_chunks)
            ]
            for cp in scatters:
                cp.wait()

    return sc_kernel, num_edges


def kernel(token_ids, edge_emb):
    seq_len = token_ids.shape[1]
    sc_call, num_edges = _build_sc_call(seq_len)
    emb_pairs = jnp.concatenate([edge_emb, edge_emb], axis=1)
    ei_flat, attr_pairs = sc_call(emb_pairs)
    edge_index = ei_flat.reshape(2, -1)[:, :num_edges]
    edge_attr = attr_pairs.reshape(num_edges, EDGE_DIM)
    return (edge_index, edge_attr)


# speedup vs baseline: 4.0688x; 4.0688x over previous
import functools

import jax
import jax.numpy as jnp
from jax import lax
from jax.experimental import pallas as pl
from jax.experimental.pallas import tpu as pltpu
from jax.experimental.pallas import tpu_sc as plsc

EDGE_DIM = 64
PAIR_DIM = 2 * EDGE_DIM
NUM_WORKERS = 32
LANES = 16
MAX_GATHER_CHUNK = 112


def _round_up(x, m):
    return (x + m - 1) // m * m


@functools.lru_cache(maxsize=None)
def _build_sc_call(seq_len):
    b0 = 2 * (seq_len - 1)
    b1 = b0 + 2 * (seq_len - 2)
    b2 = b1 + 2 * (seq_len - 3)
    num_edges = b2 + seq_len
    assert num_edges % 2 == 0
    num_pairs = num_edges // 2

    ppw = _round_up(-(-num_pairs // NUM_WORKERS), LANES)
    p_pad = ppw * NUM_WORKERS
    epw = 2 * ppw
    e_pad = 2 * p_pad

    chunk = MAX_GATHER_CHUNK
    while ppw % chunk:
        chunk -= LANES
    n_chunks = ppw // chunk
    cpr = chunk // LANES

    mesh = plsc.VectorSubcoreMesh(core_axis_name="c", subcore_axis_name="s")

    @functools.partial(
        pl.kernel,
        mesh=mesh,
        out_type=[
            jax.ShapeDtypeStruct((2 * e_pad,), jnp.int32),
            jax.ShapeDtypeStruct((num_pairs, PAIR_DIM), jnp.float32),
        ],
        scratch_types=[
            pltpu.VMEM((n_chunks, chunk), jnp.int32),
            pltpu.VMEM((n_chunks, chunk), jnp.int32),
            pltpu.VMEM((ppw, PAIR_DIM), jnp.float32),
            pltpu.VMEM((epw,), jnp.int32),
            pltpu.VMEM((epw,), jnp.int32),
            pltpu.VMEM_SHARED((5, PAIR_DIM), jnp.float32),
            pltpu.SemaphoreType.DMA,
        ],
    )
    def sc_kernel(
        emb_hbm, ei_hbm, attr_hbm, idx_v, didx_v, rows_v, src_v, dst_v,
        emb_v, sem
    ):
        @pl.when(lax.axis_index("s") == 0)
        def _():
            pltpu.sync_copy(emb_hbm, emb_v)

        plsc.subcore_barrier()
        wid = lax.axis_index("s") * 2 + lax.axis_index("c")
        pair_base = wid * ppw
        edge_base = wid * epw

        for j in range(ppw // LANES):
            pc = pair_base + j * LANES + lax.iota(jnp.int32, LANES)
            c = 2 * pc
            t = jnp.where(
                c < b0, 0, jnp.where(c < b1, 1, jnp.where(c < b2, 2, 0))
            )
            idx_v[j // cpr, pl.ds((j % cpr) * LANES, LANES)] = t
            didx_v[j // cpr, pl.ds((j % cpr) * LANES, LANES)] = jnp.minimum(
                pc, num_pairs - 1
            )

        gathers = [
            pltpu.async_copy(
                emb_v.at[idx_v.at[ch]],
                rows_v.at[pl.ds(ch * chunk, chunk)],
                sem,
            )
            for ch in range(n_chunks)
        ]

        for j in range(epw // LANES):
            c = edge_base + j * LANES + lax.iota(jnp.int32, LANES)
            in0 = c < b0
            in1 = c < b1
            in2 = c < b2
            s = jnp.where(in0, 0, jnp.where(in1, b0, jnp.where(in2, b1, b2)))
            w = jnp.where(in0, 1, jnp.where(in1, 2, jnp.where(in2, 3, 0)))
            local = c - s
            k = local >> 1
            p = local & 1
            src = jnp.where(in2, k + p * w, local)
            dst = jnp.where(in2, k + (1 - p) * w, local)
            src_v[pl.ds(j * LANES, LANES)] = src
            dst_v[pl.ds(j * LANES, LANES)] = dst

        pltpu.sync_copy(src_v, ei_hbm.at[pl.ds(edge_base, epw)])
        pltpu.sync_copy(dst_v, ei_hbm.at[pl.ds(e_pad + edge_base, epw)])

        for cp in gathers:
            cp.wait()

        @pl.when(wid < NUM_WORKERS - 1)
        def _():
            pltpu.sync_copy(rows_v, attr_hbm.at[pl.ds(pair_base, ppw)])

        @pl.when(wid == NUM_WORKERS - 1)
        def _():
            scatters = [
                pltpu.async_copy(
                    rows_v.at[pl.ds(ch * chunk, chunk)],
                    attr_hbm.at[didx_v.at[ch]],
                    sem,
                )
                for ch in range(n_chunks)
            ]
            for cp in scatters:
                cp.wait()

    return sc_kernel, num_edges


def kernel(token_ids, edge_emb):
    seq_len = token_ids.shape[1]
    sc_call, num_edges = _build_sc_call(seq_len)
    emb_pairs = jnp.concatenate([edge_emb, edge_emb], axis=1)
    ei_flat, attr_pairs = sc_call(emb_pairs)
    edge_index = ei_flat.reshape(2, -1)[:, :num_edges]
    edge_attr = attr_pairs.reshape(num_edges, EDGE_DIM)
    return (edge_index, edge_attr)


# speedup vs baseline: 4.3044x; 1.0579x over previous
import functools

import jax
import jax.numpy as jnp
from jax import lax
from jax.experimental import pallas as pl
from jax.experimental.pallas import tpu as pltpu
from jax.experimental.pallas import tpu_sc as plsc

EDGE_DIM = 64
PAIR_DIM = 2 * EDGE_DIM
NUM_TYPES_USED = 3
NUM_WORKERS = 32
LANES = 16
MAX_GATHER_CHUNK = 112


def _round_up(x, m):
    return (x + m - 1) // m * m


def _bounds(seq_len):
    b0 = 2 * (seq_len - 1)
    b1 = b0 + 2 * (seq_len - 2)
    b2 = b1 + 2 * (seq_len - 3)
    return b0, b1, b2, b2 + seq_len


@functools.lru_cache(maxsize=None)
def _build_attr_call(seq_len):
    b0, b1, b2, num_edges = _bounds(seq_len)
    assert num_edges % 2 == 0
    num_pairs = num_edges // 2

    ppw = _round_up(-(-num_pairs // NUM_WORKERS), LANES)

    chunk = MAX_GATHER_CHUNK
    while ppw % chunk:
        chunk -= LANES
    n_chunks = ppw // chunk
    cpr = chunk // LANES

    mesh = plsc.VectorSubcoreMesh(core_axis_name="c", subcore_axis_name="s")

    @functools.partial(
        pl.kernel,
        mesh=mesh,
        out_type=jax.ShapeDtypeStruct((num_pairs, PAIR_DIM), jnp.float32),
        scratch_types=[
            pltpu.VMEM((n_chunks, chunk), jnp.int32),
            pltpu.VMEM((n_chunks, chunk), jnp.int32),
            pltpu.VMEM((ppw, PAIR_DIM), jnp.float32),
            pltpu.VMEM((5, EDGE_DIM), jnp.float32),
            pltpu.VMEM((NUM_TYPES_USED, PAIR_DIM), jnp.float32),
            pltpu.VMEM_SHARED((NUM_TYPES_USED, PAIR_DIM), jnp.float32),
            pltpu.SemaphoreType.DMA,
        ],
    )
    def sc_kernel(
        emb_hbm, attr_hbm, idx_v, didx_v, rows_v, emb_v, pair_v, emb_sh, sem
    ):
        wid = lax.axis_index("s") * 2 + lax.axis_index("c")
        pair_base = wid * ppw

        @pl.when(lax.axis_index("s") == 0)
        def _():
            pltpu.sync_copy(emb_hbm, emb_v)
            for t in range(NUM_TYPES_USED):
                for i in range(EDGE_DIM // LANES):
                    v = emb_v[t, pl.ds(i * LANES, LANES)]
                    pair_v[t, pl.ds(i * LANES, LANES)] = v
                    pair_v[t, pl.ds(EDGE_DIM + i * LANES, LANES)] = v
            pltpu.sync_copy(pair_v, emb_sh)

        for j in range(ppw // LANES):
            pc = pair_base + j * LANES + lax.iota(jnp.int32, LANES)
            c = 2 * pc
            t = jnp.where(
                c < b0, 0, jnp.where(c < b1, 1, jnp.where(c < b2, 2, 0))
            )
            idx_v[j // cpr, pl.ds((j % cpr) * LANES, LANES)] = t
            didx_v[j // cpr, pl.ds((j % cpr) * LANES, LANES)] = jnp.minimum(
                pc, num_pairs - 1
            )

        plsc.subcore_barrier()

        gathers = [
            pltpu.async_copy(
                emb_sh.at[idx_v.at[ch]],
                rows_v.at[pl.ds(ch * chunk, chunk)],
                sem,
            )
            for ch in range(n_chunks)
        ]
        for cp in gathers:
            cp.wait()

        @pl.when(wid < NUM_WORKERS - 1)
        def _():
            pltpu.sync_copy(rows_v, attr_hbm.at[pl.ds(pair_base, ppw)])

        @pl.when(wid == NUM_WORKERS - 1)
        def _():
            scatters = [
                pltpu.async_copy(
                    rows_v.at[pl.ds(ch * chunk, chunk)],
                    attr_hbm.at[didx_v.at[ch]],
                    sem,
                )
                for ch in range(n_chunks)
            ]
            for cp in scatters:
                cp.wait()

    return sc_kernel


@functools.lru_cache(maxsize=None)
def _build_index_call(seq_len):
    b0, b1, b2, num_edges = _bounds(seq_len)

    def tc_kernel(out_ref):
        r = lax.broadcasted_iota(jnp.int32, (2, num_edges), 0)
        c = lax.broadcasted_iota(jnp.int32, (2, num_edges), 1)
        in0 = c < b0
        in1 = c < b1
        in2 = c < b2
        s = jnp.where(in0, 0, jnp.where(in1, b0, jnp.where(in2, b1, b2)))
        w = jnp.where(in0, 1, jnp.where(in1, 2, jnp.where(in2, 3, 0)))
        local = c - s
        k = local >> 1
        p = local & 1
        out_ref[...] = jnp.where(in2, k + (p ^ r) * w, local)

    return pl.pallas_call(
        tc_kernel,
        out_shape=jax.ShapeDtypeStruct((2, num_edges), jnp.int32),
    )


def kernel(token_ids, edge_emb):
    seq_len = token_ids.shape[1]
    num_edges = _bounds(seq_len)[3]
    attr_pairs = _build_attr_call(seq_len)(edge_emb)
    edge_index = _build_index_call(seq_len)()
    edge_attr = attr_pairs.reshape(num_edges, EDGE_DIM)
    return (edge_index, edge_attr)


# speedup vs baseline: 4.7970x; 1.1144x over previous
import functools

import jax
import jax.numpy as jnp
from jax import lax
from jax.experimental import pallas as pl
from jax.experimental.pallas import tpu as pltpu
from jax.experimental.pallas import tpu_sc as plsc

EDGE_DIM = 64
NUM_WORKERS = 32
LANES = 16
VPR = EDGE_DIM // LANES


def _round_up(x, m):
    return (x + m - 1) // m * m


def _bounds(seq_len):
    b0 = 2 * (seq_len - 1)
    b1 = b0 + 2 * (seq_len - 2)
    b2 = b1 + 2 * (seq_len - 3)
    return b0, b1, b2, b2 + seq_len


@functools.lru_cache(maxsize=None)
def _build_attr_call(seq_len):
    b0, b1, b2, num_edges = _bounds(seq_len)

    epw = _round_up(-(-num_edges // NUM_WORKERS), LANES)
    last_full = (num_edges - (NUM_WORKERS - 1) * epw) // 8 * 8
    assert 0 < last_full <= epw

    mesh = plsc.VectorSubcoreMesh(core_axis_name="c", subcore_axis_name="s")

    @functools.partial(
        pl.kernel,
        mesh=mesh,
        out_type=jax.ShapeDtypeStruct((num_edges, EDGE_DIM), jnp.float32),
        scratch_types=[
            pltpu.VMEM((5, EDGE_DIM), jnp.float32),
            pltpu.VMEM((epw, EDGE_DIM), jnp.float32),
        ],
    )
    def sc_kernel(emb_hbm, attr_hbm, emb_v, rows_v):
        wid = lax.axis_index("s") * 2 + lax.axis_index("c")
        edge_base = wid * epw

        pltpu.sync_copy(emb_hbm, emb_v)

        def type_of(c):
            return jnp.where(
                c < b0, 0, jnp.where(c < b1, 1, jnp.where(c < b2, 2, 0))
            )

        def group_body(g, carry):
            cbase = edge_base + g * LANES
            t_first = type_of(cbase)
            t_last = type_of(cbase + LANES - 1)

            @pl.when(t_first == t_last)
            def _():
                vs = [
                    emb_v[t_first, pl.ds(i * LANES, LANES)]
                    for i in range(VPR)
                ]
                for r in range(LANES):
                    for i in range(VPR):
                        rows_v[g * LANES + r, pl.ds(i * LANES, LANES)] = vs[i]

            @pl.when(t_first != t_last)
            def _():
                for r in range(LANES):
                    t_r = type_of(cbase + r)
                    for i in range(VPR):
                        rows_v[g * LANES + r, pl.ds(i * LANES, LANES)] = (
                            emb_v[t_r, pl.ds(i * LANES, LANES)]
                        )

            return carry

        lax.fori_loop(0, epw // LANES, group_body, 0)

        @pl.when(wid < NUM_WORKERS - 1)
        def _():
            pltpu.sync_copy(rows_v, attr_hbm.at[pl.ds(edge_base, epw)])

        @pl.when(wid == NUM_WORKERS - 1)
        def _():
            pltpu.sync_copy(
                rows_v.at[pl.ds(0, last_full)],
                attr_hbm.at[pl.ds(edge_base, last_full)],
            )

    return sc_kernel


@functools.lru_cache(maxsize=None)
def _build_index_call(seq_len):
    b0, b1, b2, num_edges = _bounds(seq_len)

    epw = _round_up(-(-num_edges // NUM_WORKERS), LANES)
    last_full = (num_edges - (NUM_WORKERS - 1) * epw) // 8 * 8
    tail_start = (NUM_WORKERS - 1) * epw + last_full
    assert tail_start % 8 == 0 and 0 < num_edges - tail_start < 8
    src_block = tail_start // 8 - 2
    assert src_block * 8 >= b2

    def tc_kernel(attr_in_ref, attr_out_ref, ei_ref):
        r = lax.broadcasted_iota(jnp.int32, (2, num_edges), 0)
        c = lax.broadcasted_iota(jnp.int32, (2, num_edges), 1)
        in0 = c < b0
        in1 = c < b1
        in2 = c < b2
        s = jnp.where(in0, 0, jnp.where(in1, b0, jnp.where(in2, b1, b2)))
        w = jnp.where(in0, 1, jnp.where(in1, 2, jnp.where(in2, 3, 0)))
        local = c - s
        k = local >> 1
        p = local & 1
        ei_ref[...] = jnp.where(in2, k + (p ^ r) * w, local)
        attr_out_ref[...] = attr_in_ref[...]

    return pl.pallas_call(
        tc_kernel,
        grid=(1,),
        in_specs=[
            pl.BlockSpec((8, EDGE_DIM), lambda i: (src_block, 0)),
        ],
        out_specs=[
            pl.BlockSpec((8, EDGE_DIM), lambda i: (tail_start // 8, 0)),
            pl.BlockSpec((2, num_edges), lambda i: (0, 0)),
        ],
        out_shape=[
            jax.ShapeDtypeStruct((num_edges, EDGE_DIM), jnp.float32),
            jax.ShapeDtypeStruct((2, num_edges), jnp.int32),
        ],
        input_output_aliases={0: 0},
    )


def kernel(token_ids, edge_emb):
    seq_len = token_ids.shape[1]
    attr_sc = _build_attr_call(seq_len)(edge_emb)
    edge_attr, edge_index = _build_index_call(seq_len)(attr_sc)
    return (edge_index, edge_attr)
